# baseline (speedup 1.0000x reference)
import jax
import jax.numpy as jnp
from jax import lax
from jax.experimental import pallas as pl
from jax.experimental.pallas import tpu as pltpu
from jax.experimental.pallas import tpu_sc as plsc

_B, _L, _D = 4096, 200, 64
_V = 1000000
_NW = 32
_RPW = _B // _NW
_NBUF = 2
_SPLIT = 104
_NG = _L // 16


_GATHER_DNUMS = lax.GatherDimensionNumbers(
    offset_dims=(), collapsed_slice_dims=(0,), start_index_map=(0,))


def _bcast_lane(v, lane):
    idx = jnp.full((16, 1), lane, jnp.int32)
    return lax.gather(v, idx, _GATHER_DNUMS, (1,),
                      mode=lax.GatherScatterMode.PROMISE_IN_BOUNDS)


def _encode_body(tok_hbm, tokp_hbm, table_hbm, out_hbm,
                 tok_v, tokp_v, bufs, obuf, row0_v, sem0, sem1):
    sems = (sem0, sem1)
    wid = lax.axis_index("s") * 2 + lax.axis_index("c")
    base = wid * _RPW

    pltpu.sync_copy(tok_hbm.at[pl.ds(base * _L, _RPW * _L)], tok_v)
    pltpu.sync_copy(tokp_hbm.at[pl.ds(base * _L, _RPW * _L)], tokp_v)
    pltpu.sync_copy(table_hbm.at[pl.ds(0, 1)], row0_v)
    row0 = [row0_v[0, pl.ds(d * 16, 16)] for d in range(4)]
    lanes = lax.broadcasted_iota(jnp.int32, (16,), 0)

    def fire(r, b):
        buf = bufs.at[b]
        pltpu.async_copy(table_hbm.at[tokp_v.at[pl.ds(r * _L, _SPLIT)]],
                         buf.at[pl.ds(0, _SPLIT)], sems[b])
        pltpu.async_copy(
            table_hbm.at[tokp_v.at[pl.ds(r * _L + _SPLIT, _L - _SPLIT)]],
            buf.at[pl.ds(_SPLIT, _L - _SPLIT)], sems[b])

    def process(r, b, prefetch_r):
        buf = bufs.at[b]
        pltpu.make_async_copy(table_hbm.at[pl.ds(0, _L)], buf, sems[b]).wait()

        nz = plsc.all_reduce_population_count(
            tok_v[pl.ds(r * _L, 16)] == 0)
        for k in range(1, 12):
            nz = nz + plsc.all_reduce_population_count(
                tok_v[pl.ds(r * _L + k * 16, 16)] == 0)
        tail = (tok_v[pl.ds(r * _L + _L - 16, 16)] == 0) & (lanes >= 8)
        nz = nz + plsc.all_reduce_population_count(tail)

        zero = jnp.zeros((16,), jnp.float32)

        def add_token(t, u, lsbs, accs, lane=None):
            lane = u if lane is None else lane
            sel1 = _bcast_lane(lsbs, lane) == 1
            for d in range(4):
                left = buf[t, pl.ds(d * 16, 16)]
                right = buf[t, pl.ds(_D + d * 16, 16)]
                slot = d * 2 + (u & 1)
                accs[slot] = accs[slot] + jnp.where(sel1, right, left)
            return accs

        def acc_body(g, accs):
            accs = list(accs)
            lsbs = tok_v[pl.ds(r * _L + g * 16, 16)] & 1
            for u in range(16):
                accs = add_token(g * 16 + u, u, lsbs, accs)
            return tuple(accs)

        accs = lax.fori_loop(0, _NG, acc_body, (zero,) * 8)
        accs = list(accs)
        lsbs = tok_v[pl.ds(r * _L + _L - 16, 16)] & 1
        for u in range(8):
            accs = add_token(_NG * 16 + u, u, lsbs, accs, lane=u + 8)

        if prefetch_r is not None:
            fire(prefetch_r, b)

        nzf = nz.astype(jnp.float32)
        inv = 1.0 / (_L - nz).astype(jnp.float32)
        for d in range(4):
            res = (accs[d * 2] + accs[d * 2 + 1] - nzf * row0[d]) * inv
            obuf[pl.ds(r * _D + d * 16, 16)] = res

    for b in range(_NBUF):
        fire(b, b)

    def outer(k, carry):
        for b in range(_NBUF):
            r = k * _NBUF + b
            process(r, b, r + _NBUF)
        return carry

    lax.fori_loop(0, _RPW // _NBUF - 1, outer, 0)
    for b in range(_NBUF):
        process(_RPW - _NBUF + b, b, None)

    pltpu.sync_copy(obuf, out_hbm.at[pl.ds(base * _D, _RPW * _D)])


_encoder = pl.kernel(
    _encode_body,
    out_type=jax.ShapeDtypeStruct((_B * _D,), jnp.float32),
    mesh=plsc.VectorSubcoreMesh(core_axis_name="c", subcore_axis_name="s"),
    scratch_types=[
        pltpu.VMEM((_RPW * _L,), jnp.int32),
        pltpu.VMEM((_RPW * _L,), jnp.int32),
        pltpu.VMEM((_NBUF, _L, 2 * _D), jnp.float32),
        pltpu.VMEM((_RPW * _D,), jnp.float32),
        pltpu.VMEM((1, 2 * _D), jnp.float32),
        pltpu.SemaphoreType.DMA,
        pltpu.SemaphoreType.DMA,
    ],
    compiler_params=pltpu.CompilerParams(needs_layout_passes=False),
)


@jax.jit
def kernel(token_indices, aligned_embeddings):
    tok1d = token_indices.reshape(-1)
    out = _encoder(tok1d, tok1d >> 1,
                   aligned_embeddings.reshape(_V // 2, 2 * _D))
    return out.reshape(_B, _D)
